# baseline (speedup 1.0000x reference)
import jax
import jax.numpy as jnp
from jax import lax
from jax.experimental import pallas as pl
from jax.experimental.pallas import tpu as pltpu
from jax.experimental.pallas import tpu_sc as plsc

NUM_CORES = 2
NUM_SUBCORES = 16
NW = NUM_CORES * NUM_SUBCORES


def _emb_body(x_hbm, w_hbm, out_hbm, idx_v, buf_a, buf_b, tb_a, tb_b,
              g_a, g_b, s_a, s_b):
    nct = x_hbm.shape[0]
    ib = out_hbm.shape[2]
    npw = nct // NW
    wid = lax.axis_index("s") * NUM_CORES + lax.axis_index("c")
    base = wid * npw
    pltpu.sync_copy(x_hbm.at[pl.ds(base, npw)], idx_v)

    bufs = (buf_a, buf_b)
    tbufs = (tb_a, tb_b)
    gsems = (g_a, g_b)
    ssems = (s_a, s_b)

    iota16 = lax.iota(jnp.int32, 16)
    rows = [iota16 + (g * 16) for g in range(8)]

    pltpu.async_copy(w_hbm.at[idx_v.at[0]], buf_a, g_a)

    def chunk(t, slot):
        m = base + t
        j = m // ib
        iblk = m % ib
        buf = bufs[slot]
        tbuf = tbufs[slot]

        @pl.when(t + 1 < npw)
        def _():
            pltpu.async_copy(w_hbm.at[idx_v.at[t + 1]],
                             bufs[1 - slot], gsems[1 - slot])

        pltpu.make_async_copy(w_hbm.at[idx_v.at[t]], buf, gsems[slot]).wait()

        @pl.when(t >= 2)
        def _():
            pltpu.make_async_copy(
                tbuf, out_hbm.at[0, :, 0], ssems[slot]).wait()

        def cbody(c, carry):
            cvec = jnp.full((16,), c, jnp.int32)
            clo = c % 8
            crow = c // 8
            for g in range(8):
                vals = plsc.load_gather(buf, [rows[g], cvec])
                tbuf[crow, clo, pl.ds(g * 16, 16)] = vals
            return carry

        lax.fori_loop(0, 64, cbody, 0)
        pltpu.async_copy(tbuf, out_hbm.at[j, :, iblk], ssems[slot])

    def body(i, carry):
        chunk(2 * i, 0)
        chunk(2 * i + 1, 1)
        return carry

    lax.fori_loop(0, npw // 2, body, 0)
    pltpu.make_async_copy(tb_a, out_hbm.at[0, :, 0], s_a).wait()
    pltpu.make_async_copy(tb_b, out_hbm.at[0, :, 0], s_b).wait()


def kernel(x, W):
    B, S = x.shape
    V, D = W.shape
    ib = B // 128
    nct = S * ib
    x_r = x.astype(jnp.int32).T.reshape(nct, 128)

    mesh = plsc.VectorSubcoreMesh(core_axis_name="c", subcore_axis_name="s")
    out2 = pl.kernel(
        _emb_body,
        out_type=jax.ShapeDtypeStruct((S, 8, ib, 8, 128), jnp.float32),
        mesh=mesh,
        scratch_types=[
            pltpu.VMEM((nct // NW, 128), jnp.int32),
            pltpu.VMEM((128, D), jnp.float32),
            pltpu.VMEM((128, D), jnp.float32),
            pltpu.VMEM((8, 8, 128), jnp.float32),
            pltpu.VMEM((8, 8, 128), jnp.float32),
            pltpu.SemaphoreType.DMA,
            pltpu.SemaphoreType.DMA,
            pltpu.SemaphoreType.DMA,
            pltpu.SemaphoreType.DMA,
        ],
        compiler_params=pltpu.CompilerParams(
            use_tc_tiling_on_sc=False, needs_layout_passes=False),
    )(x_r, W)

    return out2.transpose(2, 4, 0, 1, 3).reshape(B, S, D)
